# baseline (speedup 1.0000x reference)
import functools

import jax
import jax.numpy as jnp
from jax import lax
from jax.experimental import pallas as pl
from jax.experimental.pallas import tpu as pltpu
from jax.experimental.pallas import tpu_sc as plsc

VOCAB = 1000
EMBED_DIM = 64
BATCH = 1024
HIST = 200

NUM_CORES = 2
NUM_SUBCORES = 16
NW = NUM_CORES * NUM_SUBCORES
ROWS_PER_W = BATCH * HIST // NW
BATCH_PER_W = BATCH // NW
CHUNK = HIST
NCHUNK = BATCH_PER_W
NBUF = 4

_mesh = plsc.VectorSubcoreMesh(core_axis_name="c", subcore_axis_name="s")


@functools.partial(
    pl.kernel,
    out_type=jax.ShapeDtypeStruct((BATCH, HIST, EMBED_DIM), jnp.float32),
    mesh=_mesh,
    scratch_types=[
        pltpu.VMEM((NCHUNK, CHUNK), jnp.int32),
        pltpu.VMEM((NBUF, CHUNK, EMBED_DIM), jnp.float32),
        [pltpu.SemaphoreType.DMA] * NBUF,
    ],
    compiler_params=pltpu.CompilerParams(use_tc_tiling_on_sc=False),
)
def _lookup(idx_hbm, table_hbm, out_hbm, idx_v, rows_v, gsems):
    wid = lax.axis_index("s") * NUM_CORES + lax.axis_index("c")
    pltpu.sync_copy(idx_hbm.at[wid], idx_v)

    for p in range(NBUF - 1):
        pltpu.async_copy(table_hbm.at[idx_v.at[p]], rows_v.at[p], gsems[p])

    def outer(i, carry):
        for b in range(NBUF):
            j = i * NBUF + b
            nxt = j + NBUF - 1
            nb = (b + NBUF - 1) % NBUF

            @pl.when(nxt < NCHUNK)
            def _():
                pltpu.async_copy(
                    table_hbm.at[idx_v.at[nxt]], rows_v.at[nb], gsems[nb]
                )

            pltpu.make_async_copy(
                table_hbm.at[idx_v.at[j]], rows_v.at[b], gsems[b]
            ).wait()
            pltpu.sync_copy(rows_v.at[b], out_hbm.at[wid * BATCH_PER_W + j])
        return carry

    lax.fori_loop(0, NCHUNK // NBUF, outer, 0)


def kernel(token_indices, embeddings):
    idx = token_indices.reshape(NW, NCHUNK, CHUNK)
    return _lookup(idx, embeddings)
